# baseline (speedup 1.0000x reference)
import functools

import jax
import jax.numpy as jnp
from jax.experimental import pallas as pl
from jax.experimental.pallas import tpu as pltpu

_EPS = 1e-5


def _shift(v, s):
    if s == 0:
        return v
    rolled = jnp.roll(v, s, axis=1)
    lane = jax.lax.broadcasted_iota(jnp.int32, v.shape, 1)
    if s > 0:
        return jnp.where(lane < s, 0.0, rolled)
    return jnp.where(lane >= v.shape[1] + s, 0.0, rolled)


def _conv_body(*refs, fuse_bn, with_bias, with_stats, nh, in_nchw):
    i = 0
    x0, x1, x2, w_ref = refs[0], refs[1], refs[2], refs[3]
    i = 4
    if fuse_bn:
        s_ref, t_ref = refs[i], refs[i + 1]
        i += 2
    if with_bias:
        b_ref = refs[i]
        i += 1
    o_ref = refs[i]
    i += 1
    if with_stats:
        sum_ref, sq_ref = refs[i], refs[i + 1]

    b = pl.program_id(0)
    h = pl.program_id(1)

    slabs = []
    for j, xr in enumerate((x0, x1, x2)):
        v = xr[0, :, 0, 0, :] if in_nchw else xr[0, 0, :, :]
        if fuse_bn:
            v = jnp.maximum(v * s_ref[...] + t_ref[...], 0.0)
        hr = h + (j - 1)
        valid = jnp.logical_and(hr >= 0, hr < nh)
        v = v * jnp.where(valid, 1.0, 0.0).astype(v.dtype)
        slabs.append(v)
    xc = jnp.concatenate(slabs, axis=0)

    acc = None
    for jdx in range(3):
        xs = _shift(xc, 1 - jdx)
        p = jax.lax.dot(w_ref[jdx], xs, preferred_element_type=jnp.float32)
        acc = p if acc is None else acc + p
    if with_bias:
        acc = acc + b_ref[...]
    o_ref[0, 0, :, :] = acc

    if with_stats:
        @pl.when(jnp.logical_and(b == 0, h == 0))
        def _init():
            sum_ref[...] = jnp.zeros_like(sum_ref)
            sq_ref[...] = jnp.zeros_like(sq_ref)

        sum_ref[...] += acc
        sq_ref[...] += acc * acc


def _xim(b, h, *, j, nh):
    return (b, jnp.clip(h + j - 1, 0, nh - 1), 0, 0)


def _xim_nchw(b, h, *, j, nh):
    return (b, 0, jnp.clip(h + j - 1, 0, nh - 1), 0, 0)


def _conv(x, wcat, s=None, t=None, bias=None, with_stats=False, in_nchw=False):
    if in_nchw:
        B, cin, nh, wd = x.shape
        x = x.reshape(B, cin, nh, 1, wd)
        xblock = (1, cin, 1, 1, wd)
        xmap = _xim_nchw
    else:
        B, nh, cin, wd = x.shape
        xblock = (1, 1, cin, wd)
        xmap = _xim
    cout = wcat.shape[1]
    fuse_bn = s is not None
    with_bias = bias is not None

    in_specs = [
        pl.BlockSpec(xblock, functools.partial(xmap, j=j, nh=nh))
        for j in range(3)
    ]
    in_specs.append(pl.BlockSpec(wcat.shape, lambda b, h: (0, 0, 0)))
    operands = [x, x, x, wcat]
    if fuse_bn:
        in_specs += [pl.BlockSpec(s.shape, lambda b, h: (0, 0)),
                     pl.BlockSpec(t.shape, lambda b, h: (0, 0))]
        operands += [s, t]
    if with_bias:
        in_specs.append(pl.BlockSpec(bias.shape, lambda b, h: (0, 0)))
        operands.append(bias)

    out_shape = [jax.ShapeDtypeStruct((B, nh, cout, wd), jnp.float32)]
    out_specs = [pl.BlockSpec((1, 1, cout, wd), lambda b, h: (b, h, 0, 0))]
    if with_stats:
        out_shape += [jax.ShapeDtypeStruct((cout, wd), jnp.float32)] * 2
        out_specs += [pl.BlockSpec((cout, wd), lambda b, h: (0, 0))] * 2

    body = functools.partial(_conv_body, fuse_bn=fuse_bn, with_bias=with_bias,
                             with_stats=with_stats, nh=nh, in_nchw=in_nchw)
    res = pl.pallas_call(
        body,
        grid=(B, nh),
        in_specs=in_specs,
        out_specs=out_specs,
        out_shape=out_shape,
        compiler_params=pltpu.CompilerParams(
            dimension_semantics=("arbitrary", "arbitrary")),
    )(*operands)
    return res


def _prep_w(w):
    return jnp.transpose(w, (3, 0, 2, 1)).reshape(3, w.shape[0], 3 * w.shape[1])


def _bn_fold(sum_o, sq_o, g, b, n, wd):
    m = jnp.sum(sum_o, axis=1, keepdims=True) / n
    v = jnp.sum(sq_o, axis=1, keepdims=True) / n - m * m
    s = g.reshape(-1, 1) * jax.lax.rsqrt(v + _EPS)
    t = b.reshape(-1, 1) - m * s
    c = s.shape[0]
    return jnp.broadcast_to(s, (c, wd)), jnp.broadcast_to(t, (c, wd))


def kernel(feats, shared_w, shared_bn_g, shared_bn_b,
           reg_w1, reg_bn_g, reg_bn_b, reg_w2, reg_b2,
           height_w1, height_bn_g, height_bn_b, height_w2, height_b2,
           dim_w1, dim_bn_g, dim_bn_b, dim_w2, dim_b2,
           rot_w1, rot_bn_g, rot_bn_b, rot_w2, rot_b2,
           vel_w1, vel_bn_g, vel_bn_b, vel_w2, vel_b2,
           heatmap_w1, heatmap_bn_g, heatmap_bn_b, heatmap_w2, heatmap_b2):
    x = feats[0]
    B, _, nh, wd = x.shape
    n = B * nh * wd

    y, s1, q1 = _conv(x, _prep_w(shared_w), with_stats=True, in_nchw=True)
    sc1, tc1 = _bn_fold(s1, q1, shared_bn_g, shared_bn_b, n, wd)

    w1 = jnp.concatenate(
        [reg_w1, height_w1, dim_w1, rot_w1, vel_w1, heatmap_w1], axis=0)
    hh, s2, q2 = _conv(y, _prep_w(w1), s=sc1, t=tc1, with_stats=True)
    g2 = jnp.concatenate(
        [reg_bn_g, height_bn_g, dim_bn_g, rot_bn_g, vel_bn_g, heatmap_bn_g])
    bb2 = jnp.concatenate(
        [reg_bn_b, height_bn_b, dim_bn_b, rot_bn_b, vel_bn_b, heatmap_bn_b])
    sc2, tc2 = _bn_fold(s2, q2, g2, bb2, n, wd)

    ocs = (2, 1, 3, 2, 2, 1)
    w2s = (reg_w2, height_w2, dim_w2, rot_w2, vel_w2, heatmap_w2)
    b2s = (reg_b2, height_b2, dim_b2, rot_b2, vel_b2, heatmap_b2)
    c1 = w2s[0].shape[1]
    wbd = jnp.zeros((16, 6 * c1, 3, 3), jnp.float32)
    r = 0
    for k, (oc, w2) in enumerate(zip(ocs, w2s)):
        wbd = wbd.at[r:r + oc, c1 * k:c1 * (k + 1)].set(w2)
        r += oc
    bias = jnp.pad(jnp.concatenate(b2s).reshape(-1, 1), ((0, 16 - r), (0, 0)))
    bias = jnp.broadcast_to(bias, (16, wd))

    out3 = _conv(hh, _prep_w(wbd), s=sc2, t=tc2, bias=bias)[0]
    out3 = jnp.transpose(out3, (0, 2, 1, 3))

    outs = []
    r = 0
    for oc in ocs:
        outs.append(out3[:, r:r + oc])
        r += oc
    return tuple(outs)


# speedup vs baseline: 1.0984x; 1.0984x over previous
import functools

import jax
import jax.numpy as jnp
from jax.experimental import pallas as pl
from jax.experimental.pallas import tpu as pltpu

_EPS = 1e-5


def _shift(v, s):
    if s == 0:
        return v
    rolled = jnp.roll(v, s, axis=1)
    lane = jax.lax.broadcasted_iota(jnp.int32, v.shape, 1)
    if s > 0:
        return jnp.where(lane < s, 0.0, rolled)
    return jnp.where(lane >= v.shape[1] + s, 0.0, rolled)


def _conv_body(*refs, fuse_bn, with_bias, with_stats, nh, in_nchw):
    i = 0
    x0, x1, x2, w_ref = refs[0], refs[1], refs[2], refs[3]
    i = 4
    if fuse_bn:
        s_ref, t_ref = refs[i], refs[i + 1]
        i += 2
    if with_bias:
        b_ref = refs[i]
        i += 1
    o_ref = refs[i]
    i += 1
    if with_stats:
        sum_ref, sq_ref = refs[i], refs[i + 1]

    b = pl.program_id(0)
    h = pl.program_id(1)

    slabs = []
    for j, xr in enumerate((x0, x1, x2)):
        v = xr[0, :, 0, 0, :] if in_nchw else xr[0, 0, :, :]
        if fuse_bn:
            v = jnp.maximum(v.astype(jnp.float32) * s_ref[...] + t_ref[...],
                            0.0)
        hr = h + (j - 1)
        valid = jnp.logical_and(hr >= 0, hr < nh)
        v = v * jnp.where(valid, 1.0, 0.0).astype(v.dtype)
        slabs.append(v.astype(jnp.bfloat16))
    xc = jnp.concatenate(slabs, axis=0)

    acc = None
    for jdx in range(3):
        xs = _shift(xc, 1 - jdx)
        p = jax.lax.dot(w_ref[jdx], xs, preferred_element_type=jnp.float32)
        acc = p if acc is None else acc + p
    if with_bias:
        acc = acc + b_ref[...]
    o_ref[0, 0, :, :] = acc.astype(o_ref.dtype)

    if with_stats:
        @pl.when(jnp.logical_and(b == 0, h == 0))
        def _init():
            sum_ref[...] = jnp.zeros_like(sum_ref)
            sq_ref[...] = jnp.zeros_like(sq_ref)

        sum_ref[...] += acc
        sq_ref[...] += acc * acc


def _xim(b, h, *, j, nh):
    return (b, jnp.clip(h + j - 1, 0, nh - 1), 0, 0)


def _xim_nchw(b, h, *, j, nh):
    return (b, 0, jnp.clip(h + j - 1, 0, nh - 1), 0, 0)


def _conv(x, wcat, s=None, t=None, bias=None, with_stats=False, in_nchw=False,
          out_dtype=jnp.float32):
    if in_nchw:
        B, cin, nh, wd = x.shape
        x = x.reshape(B, cin, nh, 1, wd)
        xblock = (1, cin, 1, 1, wd)
        xmap = _xim_nchw
    else:
        B, nh, cin, wd = x.shape
        xblock = (1, 1, cin, wd)
        xmap = _xim
    cout = wcat.shape[1]
    fuse_bn = s is not None
    with_bias = bias is not None

    in_specs = [
        pl.BlockSpec(xblock, functools.partial(xmap, j=j, nh=nh))
        for j in range(3)
    ]
    in_specs.append(pl.BlockSpec(wcat.shape, lambda b, h: (0, 0, 0)))
    operands = [x, x, x, wcat]
    if fuse_bn:
        in_specs += [pl.BlockSpec(s.shape, lambda b, h: (0, 0)),
                     pl.BlockSpec(t.shape, lambda b, h: (0, 0))]
        operands += [s, t]
    if with_bias:
        in_specs.append(pl.BlockSpec(bias.shape, lambda b, h: (0, 0)))
        operands.append(bias)

    out_shape = [jax.ShapeDtypeStruct((B, nh, cout, wd), out_dtype)]
    out_specs = [pl.BlockSpec((1, 1, cout, wd), lambda b, h: (b, h, 0, 0))]
    if with_stats:
        out_shape += [jax.ShapeDtypeStruct((cout, wd), jnp.float32)] * 2
        out_specs += [pl.BlockSpec((cout, wd), lambda b, h: (0, 0))] * 2

    body = functools.partial(_conv_body, fuse_bn=fuse_bn, with_bias=with_bias,
                             with_stats=with_stats, nh=nh, in_nchw=in_nchw)
    res = pl.pallas_call(
        body,
        grid=(B, nh),
        in_specs=in_specs,
        out_specs=out_specs,
        out_shape=out_shape,
        compiler_params=pltpu.CompilerParams(
            dimension_semantics=("arbitrary", "arbitrary")),
    )(*operands)
    return res


def _prep_w(w):
    return jnp.transpose(w, (3, 0, 2, 1)).reshape(3, w.shape[0], 3 * w.shape[1])


def _bn_fold(sum_o, sq_o, g, b, n, wd):
    m = jnp.sum(sum_o, axis=1, keepdims=True) / n
    v = jnp.sum(sq_o, axis=1, keepdims=True) / n - m * m
    s = g.reshape(-1, 1) * jax.lax.rsqrt(v + _EPS)
    t = b.reshape(-1, 1) - m * s
    c = s.shape[0]
    return jnp.broadcast_to(s, (c, wd)), jnp.broadcast_to(t, (c, wd))


def kernel(feats, shared_w, shared_bn_g, shared_bn_b,
           reg_w1, reg_bn_g, reg_bn_b, reg_w2, reg_b2,
           height_w1, height_bn_g, height_bn_b, height_w2, height_b2,
           dim_w1, dim_bn_g, dim_bn_b, dim_w2, dim_b2,
           rot_w1, rot_bn_g, rot_bn_b, rot_w2, rot_b2,
           vel_w1, vel_bn_g, vel_bn_b, vel_w2, vel_b2,
           heatmap_w1, heatmap_bn_g, heatmap_bn_b, heatmap_w2, heatmap_b2):
    x = jnp.transpose(feats[0], (0, 2, 1, 3)).astype(jnp.bfloat16)
    B, nh, _, wd = x.shape
    n = B * nh * wd

    y, s1, q1 = _conv(x, _prep_w(shared_w).astype(jnp.bfloat16),
                      with_stats=True)
    sc1, tc1 = _bn_fold(s1, q1, shared_bn_g, shared_bn_b, n, wd)

    w1 = jnp.concatenate(
        [reg_w1, height_w1, dim_w1, rot_w1, vel_w1, heatmap_w1], axis=0)
    hh, s2, q2 = _conv(y, _prep_w(w1).astype(jnp.bfloat16), s=sc1, t=tc1,
                       with_stats=True, out_dtype=jnp.bfloat16)
    g2 = jnp.concatenate(
        [reg_bn_g, height_bn_g, dim_bn_g, rot_bn_g, vel_bn_g, heatmap_bn_g])
    bb2 = jnp.concatenate(
        [reg_bn_b, height_bn_b, dim_bn_b, rot_bn_b, vel_bn_b, heatmap_bn_b])
    sc2, tc2 = _bn_fold(s2, q2, g2, bb2, n, wd)

    ocs = (2, 1, 3, 2, 2, 1)
    w2s = (reg_w2, height_w2, dim_w2, rot_w2, vel_w2, heatmap_w2)
    b2s = (reg_b2, height_b2, dim_b2, rot_b2, vel_b2, heatmap_b2)
    c1 = w2s[0].shape[1]
    wbd = jnp.zeros((16, 6 * c1, 3, 3), jnp.float32)
    r = 0
    for k, (oc, w2) in enumerate(zip(ocs, w2s)):
        wbd = wbd.at[r:r + oc, c1 * k:c1 * (k + 1)].set(w2)
        r += oc
    bias = jnp.pad(jnp.concatenate(b2s).reshape(-1, 1), ((0, 16 - r), (0, 0)))
    bias = jnp.broadcast_to(bias, (16, wd))

    out3 = _conv(hh, _prep_w(wbd).astype(jnp.bfloat16), s=sc2, t=tc2,
                 bias=bias)[0]
    out3 = jnp.transpose(out3, (0, 2, 1, 3))

    outs = []
    r = 0
    for oc in ocs:
        outs.append(out3[:, r:r + oc])
        r += oc
    return tuple(outs)


# speedup vs baseline: 3.4177x; 3.1116x over previous
import functools

import jax
import jax.numpy as jnp
from jax.experimental import pallas as pl
from jax.experimental.pallas import tpu as pltpu

_EPS = 1e-5
_ROWS = 8


def _shift(v, s, blk):
    if s == 0:
        return v
    rolled = jnp.roll(v, s, axis=1)
    lane = jnp.bitwise_and(
        jax.lax.broadcasted_iota(jnp.int32, v.shape, 1), blk - 1)
    if s > 0:
        return jnp.where(lane < s, 0.0, rolled)
    return jnp.where(lane >= blk + s, 0.0, rolled)


def _conv_body(*refs, fuse_bn, with_bias, with_stats, nh, rows):
    x_refs = refs[0:3]
    w_ref = refs[3]
    i = 4
    if fuse_bn:
        s_ref, t_ref = refs[i], refs[i + 1]
        i += 2
    if with_bias:
        b_ref = refs[i]
        i += 1
    o_ref = refs[i]
    i += 1
    if with_stats:
        sum_ref, sq_ref = refs[i], refs[i + 1]

    b = pl.program_id(0)
    h = pl.program_id(1)

    if with_stats:
        @pl.when(jnp.logical_and(b == 0, h == 0))
        def _init():
            sum_ref[...] = jnp.zeros_like(sum_ref)
            sq_ref[...] = jnp.zeros_like(sq_ref)

    wd = o_ref.shape[3]
    slabs = []
    for L in range(rows - 1, 2 * rows + 1):
        v = x_refs[L // rows][0, L % rows, :, :]
        if fuse_bn:
            v = jnp.maximum(v.astype(jnp.float32) * s_ref[...] + t_ref[...],
                            0.0)
        gi = h * rows + (L - rows)
        valid = jnp.logical_and(gi >= 0, gi < nh)
        v = v * jnp.where(valid, 1.0, 0.0).astype(v.dtype)
        slabs.append(v.astype(jnp.bfloat16))
    slab = jnp.concatenate(slabs, axis=1)

    xk = jnp.concatenate(
        [slab[:, d * wd:(d + rows) * wd] for d in range(3)], axis=0)
    acc = None
    for jdx in range(3):
        xs = _shift(xk, 1 - jdx, wd)
        p = jax.lax.dot(w_ref[jdx], xs,
                        preferred_element_type=jnp.float32)
        acc = p if acc is None else acc + p
    if with_bias:
        acc = acc + jnp.tile(b_ref[...], (1, rows))
    for r in range(rows):
        o_ref[0, r, :, :] = acc[:, r * wd:(r + 1) * wd].astype(o_ref.dtype)
    if with_stats:
        sum_ref[...] += acc
        sq_ref[...] += acc * acc


def _xim(b, h, *, j, nt):
    return (b, jnp.clip(h + j - 1, 0, nt - 1), 0, 0)


def _conv(x, wcat, s=None, t=None, bias=None, with_stats=False,
          out_dtype=jnp.float32):
    B, nh, cin, wd = x.shape
    cout = wcat.shape[1]
    fuse_bn = s is not None
    with_bias = bias is not None
    rows = _ROWS
    nt = nh // rows

    in_specs = [
        pl.BlockSpec((1, rows, cin, wd), functools.partial(_xim, j=j, nt=nt))
        for j in range(3)
    ]
    in_specs.append(pl.BlockSpec(wcat.shape, lambda b, h: (0, 0, 0)))
    operands = [x, x, x, wcat]
    if fuse_bn:
        in_specs += [pl.BlockSpec(s.shape, lambda b, h: (0, 0)),
                     pl.BlockSpec(t.shape, lambda b, h: (0, 0))]
        operands += [s, t]
    if with_bias:
        in_specs.append(pl.BlockSpec(bias.shape, lambda b, h: (0, 0)))
        operands.append(bias)

    out_shape = [jax.ShapeDtypeStruct((B, nh, cout, wd), out_dtype)]
    out_specs = [pl.BlockSpec((1, rows, cout, wd), lambda b, h: (b, h, 0, 0))]
    if with_stats:
        out_shape += [jax.ShapeDtypeStruct((cout, rows * wd), jnp.float32)] * 2
        out_specs += [pl.BlockSpec((cout, rows * wd), lambda b, h: (0, 0))] * 2

    body = functools.partial(_conv_body, fuse_bn=fuse_bn, with_bias=with_bias,
                             with_stats=with_stats, nh=nh, rows=rows)
    res = pl.pallas_call(
        body,
        grid=(B, nt),
        in_specs=in_specs,
        out_specs=out_specs,
        out_shape=out_shape,
        compiler_params=pltpu.CompilerParams(
            dimension_semantics=("arbitrary", "arbitrary")),
    )(*operands)
    return res


def _prep_w(w):
    wt = jnp.transpose(w, (3, 0, 2, 1)).reshape(3, w.shape[0], 3 * w.shape[1])
    return wt.astype(jnp.bfloat16)


def _bn_fold(sum_o, sq_o, g, b, n, wd):
    m = jnp.sum(sum_o, axis=1, keepdims=True) / n
    v = jnp.sum(sq_o, axis=1, keepdims=True) / n - m * m
    s = g.reshape(-1, 1) * jax.lax.rsqrt(v + _EPS)
    t = b.reshape(-1, 1) - m * s
    c = s.shape[0]
    return jnp.broadcast_to(s, (c, wd)), jnp.broadcast_to(t, (c, wd))


def kernel(feats, shared_w, shared_bn_g, shared_bn_b,
           reg_w1, reg_bn_g, reg_bn_b, reg_w2, reg_b2,
           height_w1, height_bn_g, height_bn_b, height_w2, height_b2,
           dim_w1, dim_bn_g, dim_bn_b, dim_w2, dim_b2,
           rot_w1, rot_bn_g, rot_bn_b, rot_w2, rot_b2,
           vel_w1, vel_bn_g, vel_bn_b, vel_w2, vel_b2,
           heatmap_w1, heatmap_bn_g, heatmap_bn_b, heatmap_w2, heatmap_b2):
    x = jnp.transpose(feats[0], (0, 2, 1, 3)).astype(jnp.bfloat16)
    B, nh, _, wd = x.shape
    n = B * nh * wd

    y, s1, q1 = _conv(x, _prep_w(shared_w), with_stats=True,
                      out_dtype=jnp.bfloat16)
    sc1, tc1 = _bn_fold(s1, q1, shared_bn_g, shared_bn_b, n, wd)

    w1 = jnp.concatenate(
        [reg_w1, height_w1, dim_w1, rot_w1, vel_w1, heatmap_w1], axis=0)
    hh, s2, q2 = _conv(y, _prep_w(w1), s=sc1, t=tc1, with_stats=True,
                       out_dtype=jnp.bfloat16)
    g2 = jnp.concatenate(
        [reg_bn_g, height_bn_g, dim_bn_g, rot_bn_g, vel_bn_g, heatmap_bn_g])
    bb2 = jnp.concatenate(
        [reg_bn_b, height_bn_b, dim_bn_b, rot_bn_b, vel_bn_b, heatmap_bn_b])
    sc2, tc2 = _bn_fold(s2, q2, g2, bb2, n, wd)

    ocs = (2, 1, 3, 2, 2, 1)
    w2s = (reg_w2, height_w2, dim_w2, rot_w2, vel_w2, heatmap_w2)
    b2s = (reg_b2, height_b2, dim_b2, rot_b2, vel_b2, heatmap_b2)
    c1 = w2s[0].shape[1]
    wbd = jnp.zeros((16, 6 * c1, 3, 3), jnp.float32)
    r = 0
    for k, (oc, w2) in enumerate(zip(ocs, w2s)):
        wbd = wbd.at[r:r + oc, c1 * k:c1 * (k + 1)].set(w2)
        r += oc
    bias = jnp.pad(jnp.concatenate(b2s).reshape(-1, 1), ((0, 16 - r), (0, 0)))
    bias = jnp.broadcast_to(bias, (16, wd))

    out3 = _conv(hh, _prep_w(wbd), s=sc2, t=tc2, bias=bias)[0]
    out3 = jnp.transpose(out3, (0, 2, 1, 3))

    outs = []
    r = 0
    for oc in ocs:
        outs.append(out3[:, r:r + oc])
        r += oc
    return tuple(outs)


# speedup vs baseline: 3.8702x; 1.1324x over previous
import functools

import jax
import jax.numpy as jnp
from jax.experimental import pallas as pl
from jax.experimental.pallas import tpu as pltpu

_EPS = 1e-5
_ROWS = 16


def _shift(v, s, blk):
    if s == 0:
        return v
    rolled = jnp.roll(v, s, axis=1)
    lane = jnp.bitwise_and(
        jax.lax.broadcasted_iota(jnp.int32, v.shape, 1), blk - 1)
    if s > 0:
        return jnp.where(lane < s, 0.0, rolled)
    return jnp.where(lane >= blk + s, 0.0, rolled)


def _conv_body(*refs, fuse_bn, with_bias, with_stats, nh, rows):
    x_refs = refs[0:3]
    w_ref = refs[3]
    i = 4
    if fuse_bn:
        s_ref, t_ref = refs[i], refs[i + 1]
        i += 2
    if with_bias:
        b_ref = refs[i]
        i += 1
    o_ref = refs[i]
    i += 1
    if with_stats:
        sum_ref, sq_ref = refs[i], refs[i + 1]

    b = pl.program_id(0)
    h = pl.program_id(1)

    if with_stats:
        @pl.when(jnp.logical_and(b == 0, h == 0))
        def _init():
            sum_ref[...] = jnp.zeros_like(sum_ref)
            sq_ref[...] = jnp.zeros_like(sq_ref)

    wd = o_ref.shape[3]
    slabs = []
    for L in range(rows - 1, 2 * rows + 1):
        v = x_refs[L // rows][0, L % rows, :, :]
        if fuse_bn:
            v = jnp.maximum(v.astype(jnp.float32) * s_ref[...] + t_ref[...],
                            0.0)
        gi = h * rows + (L - rows)
        valid = jnp.logical_and(gi >= 0, gi < nh)
        v = v * jnp.where(valid, 1.0, 0.0).astype(v.dtype)
        slabs.append(v.astype(jnp.bfloat16))
    slab = jnp.concatenate(slabs, axis=1)

    xk = jnp.concatenate(
        [slab[:, d * wd:(d + rows) * wd] for d in range(3)], axis=0)
    acc = None
    for jdx in range(3):
        xs = _shift(xk, 1 - jdx, wd)
        p = jax.lax.dot(w_ref[jdx], xs,
                        preferred_element_type=jnp.float32)
        acc = p if acc is None else acc + p
    if with_bias:
        acc = acc + jnp.tile(b_ref[...], (1, rows))
    for r in range(rows):
        o_ref[0, r, :, :] = acc[:, r * wd:(r + 1) * wd].astype(o_ref.dtype)
    if with_stats:
        sum_ref[...] += acc
        sq_ref[...] += acc * acc


def _xim(b, h, *, j, nt):
    return (b, jnp.clip(h + j - 1, 0, nt - 1), 0, 0)


def _conv(x, wcat, s=None, t=None, bias=None, with_stats=False,
          out_dtype=jnp.float32):
    B, nh, cin, wd = x.shape
    cout = wcat.shape[1]
    fuse_bn = s is not None
    with_bias = bias is not None
    rows = _ROWS
    nt = nh // rows

    in_specs = [
        pl.BlockSpec((1, rows, cin, wd), functools.partial(_xim, j=j, nt=nt))
        for j in range(3)
    ]
    in_specs.append(pl.BlockSpec(wcat.shape, lambda b, h: (0, 0, 0)))
    operands = [x, x, x, wcat]
    if fuse_bn:
        in_specs += [pl.BlockSpec(s.shape, lambda b, h: (0, 0)),
                     pl.BlockSpec(t.shape, lambda b, h: (0, 0))]
        operands += [s, t]
    if with_bias:
        in_specs.append(pl.BlockSpec(bias.shape, lambda b, h: (0, 0)))
        operands.append(bias)

    out_shape = [jax.ShapeDtypeStruct((B, nh, cout, wd), out_dtype)]
    out_specs = [pl.BlockSpec((1, rows, cout, wd), lambda b, h: (b, h, 0, 0))]
    if with_stats:
        out_shape += [jax.ShapeDtypeStruct((cout, rows * wd), jnp.float32)] * 2
        out_specs += [pl.BlockSpec((cout, rows * wd), lambda b, h: (0, 0))] * 2

    body = functools.partial(_conv_body, fuse_bn=fuse_bn, with_bias=with_bias,
                             with_stats=with_stats, nh=nh, rows=rows)
    res = pl.pallas_call(
        body,
        grid=(B, nt),
        in_specs=in_specs,
        out_specs=out_specs,
        out_shape=out_shape,
        compiler_params=pltpu.CompilerParams(
            dimension_semantics=("arbitrary", "arbitrary")),
    )(*operands)
    return res


def _prep_w(w):
    wt = jnp.transpose(w, (3, 0, 2, 1)).reshape(3, w.shape[0], 3 * w.shape[1])
    return wt.astype(jnp.bfloat16)


def _bn_fold(sum_o, sq_o, g, b, n, wd):
    m = jnp.sum(sum_o, axis=1, keepdims=True) / n
    v = jnp.sum(sq_o, axis=1, keepdims=True) / n - m * m
    s = g.reshape(-1, 1) * jax.lax.rsqrt(v + _EPS)
    t = b.reshape(-1, 1) - m * s
    c = s.shape[0]
    return jnp.broadcast_to(s, (c, wd)), jnp.broadcast_to(t, (c, wd))


def kernel(feats, shared_w, shared_bn_g, shared_bn_b,
           reg_w1, reg_bn_g, reg_bn_b, reg_w2, reg_b2,
           height_w1, height_bn_g, height_bn_b, height_w2, height_b2,
           dim_w1, dim_bn_g, dim_bn_b, dim_w2, dim_b2,
           rot_w1, rot_bn_g, rot_bn_b, rot_w2, rot_b2,
           vel_w1, vel_bn_g, vel_bn_b, vel_w2, vel_b2,
           heatmap_w1, heatmap_bn_g, heatmap_bn_b, heatmap_w2, heatmap_b2):
    x = jnp.transpose(feats[0], (0, 2, 1, 3)).astype(jnp.bfloat16)
    B, nh, _, wd = x.shape
    n = B * nh * wd

    y, s1, q1 = _conv(x, _prep_w(shared_w), with_stats=True,
                      out_dtype=jnp.bfloat16)
    sc1, tc1 = _bn_fold(s1, q1, shared_bn_g, shared_bn_b, n, wd)

    w1 = jnp.concatenate(
        [reg_w1, height_w1, dim_w1, rot_w1, vel_w1, heatmap_w1], axis=0)
    hh, s2, q2 = _conv(y, _prep_w(w1), s=sc1, t=tc1, with_stats=True,
                       out_dtype=jnp.bfloat16)
    g2 = jnp.concatenate(
        [reg_bn_g, height_bn_g, dim_bn_g, rot_bn_g, vel_bn_g, heatmap_bn_g])
    bb2 = jnp.concatenate(
        [reg_bn_b, height_bn_b, dim_bn_b, rot_bn_b, vel_bn_b, heatmap_bn_b])
    sc2, tc2 = _bn_fold(s2, q2, g2, bb2, n, wd)

    ocs = (2, 1, 3, 2, 2, 1)
    w2s = (reg_w2, height_w2, dim_w2, rot_w2, vel_w2, heatmap_w2)
    b2s = (reg_b2, height_b2, dim_b2, rot_b2, vel_b2, heatmap_b2)
    c1 = w2s[0].shape[1]
    wbd = jnp.zeros((16, 6 * c1, 3, 3), jnp.float32)
    r = 0
    for k, (oc, w2) in enumerate(zip(ocs, w2s)):
        wbd = wbd.at[r:r + oc, c1 * k:c1 * (k + 1)].set(w2)
        r += oc
    bias = jnp.pad(jnp.concatenate(b2s).reshape(-1, 1), ((0, 16 - r), (0, 0)))
    bias = jnp.broadcast_to(bias, (16, wd))

    out3 = _conv(hh, _prep_w(wbd), s=sc2, t=tc2, bias=bias)[0]
    out3 = jnp.transpose(out3, (0, 2, 1, 3))

    outs = []
    r = 0
    for oc in ocs:
        outs.append(out3[:, r:r + oc])
        r += oc
    return tuple(outs)


# speedup vs baseline: 3.9207x; 1.0130x over previous
import functools

import jax
import jax.numpy as jnp
from jax.experimental import pallas as pl
from jax.experimental.pallas import tpu as pltpu

_EPS = 1e-5
_ROWS = 32


def _shift(v, s, blk):
    if s == 0:
        return v
    rolled = jnp.roll(v, s, axis=1)
    lane = jnp.bitwise_and(
        jax.lax.broadcasted_iota(jnp.int32, v.shape, 1), blk - 1)
    if s > 0:
        return jnp.where(lane < s, 0.0, rolled)
    return jnp.where(lane >= blk + s, 0.0, rolled)


def _conv_body(*refs, fuse_bn, with_bias, with_stats, nh, rows):
    x_refs = refs[0:3]
    w_ref = refs[3]
    i = 4
    if fuse_bn:
        s_ref, t_ref = refs[i], refs[i + 1]
        i += 2
    if with_bias:
        b_ref = refs[i]
        i += 1
    o_ref = refs[i]
    i += 1
    if with_stats:
        sum_ref, sq_ref = refs[i], refs[i + 1]

    b = pl.program_id(0)
    h = pl.program_id(1)

    if with_stats:
        @pl.when(jnp.logical_and(b == 0, h == 0))
        def _init():
            sum_ref[...] = jnp.zeros_like(sum_ref)
            sq_ref[...] = jnp.zeros_like(sq_ref)

    wd = o_ref.shape[3]
    slabs = []
    for L in range(rows - 1, 2 * rows + 1):
        v = x_refs[L // rows][0, L % rows, :, :]
        if fuse_bn:
            v = jnp.maximum(v.astype(jnp.float32) * s_ref[...] + t_ref[...],
                            0.0)
        gi = h * rows + (L - rows)
        valid = jnp.logical_and(gi >= 0, gi < nh)
        v = v * jnp.where(valid, 1.0, 0.0).astype(v.dtype)
        slabs.append(v.astype(jnp.bfloat16))
    slab = jnp.concatenate(slabs, axis=1)

    xk = jnp.concatenate(
        [slab[:, d * wd:(d + rows) * wd] for d in range(3)], axis=0)
    acc = None
    for jdx in range(3):
        xs = _shift(xk, 1 - jdx, wd)
        p = jax.lax.dot(w_ref[jdx], xs,
                        preferred_element_type=jnp.float32)
        acc = p if acc is None else acc + p
    if with_bias:
        acc = acc + jnp.tile(b_ref[...], (1, rows))
    for r in range(rows):
        o_ref[0, r, :, :] = acc[:, r * wd:(r + 1) * wd].astype(o_ref.dtype)
    if with_stats:
        sum_ref[...] += acc
        sq_ref[...] += acc * acc


def _xim(b, h, *, j, nt):
    return (b, jnp.clip(h + j - 1, 0, nt - 1), 0, 0)


def _conv(x, wcat, s=None, t=None, bias=None, with_stats=False,
          out_dtype=jnp.float32):
    B, nh, cin, wd = x.shape
    cout = wcat.shape[1]
    fuse_bn = s is not None
    with_bias = bias is not None
    rows = _ROWS
    nt = nh // rows

    in_specs = [
        pl.BlockSpec((1, rows, cin, wd), functools.partial(_xim, j=j, nt=nt))
        for j in range(3)
    ]
    in_specs.append(pl.BlockSpec(wcat.shape, lambda b, h: (0, 0, 0)))
    operands = [x, x, x, wcat]
    if fuse_bn:
        in_specs += [pl.BlockSpec(s.shape, lambda b, h: (0, 0)),
                     pl.BlockSpec(t.shape, lambda b, h: (0, 0))]
        operands += [s, t]
    if with_bias:
        in_specs.append(pl.BlockSpec(bias.shape, lambda b, h: (0, 0)))
        operands.append(bias)

    out_shape = [jax.ShapeDtypeStruct((B, nh, cout, wd), out_dtype)]
    out_specs = [pl.BlockSpec((1, rows, cout, wd), lambda b, h: (b, h, 0, 0))]
    if with_stats:
        out_shape += [jax.ShapeDtypeStruct((cout, rows * wd), jnp.float32)] * 2
        out_specs += [pl.BlockSpec((cout, rows * wd), lambda b, h: (0, 0))] * 2

    body = functools.partial(_conv_body, fuse_bn=fuse_bn, with_bias=with_bias,
                             with_stats=with_stats, nh=nh, rows=rows)
    res = pl.pallas_call(
        body,
        grid=(B, nt),
        in_specs=in_specs,
        out_specs=out_specs,
        out_shape=out_shape,
        compiler_params=pltpu.CompilerParams(
            dimension_semantics=("arbitrary", "arbitrary")),
    )(*operands)
    return res


def _prep_w(w):
    wt = jnp.transpose(w, (3, 0, 2, 1)).reshape(3, w.shape[0], 3 * w.shape[1])
    return wt.astype(jnp.bfloat16)


def _bn_fold(sum_o, sq_o, g, b, n, wd):
    m = jnp.sum(sum_o, axis=1, keepdims=True) / n
    v = jnp.sum(sq_o, axis=1, keepdims=True) / n - m * m
    s = g.reshape(-1, 1) * jax.lax.rsqrt(v + _EPS)
    t = b.reshape(-1, 1) - m * s
    c = s.shape[0]
    return jnp.broadcast_to(s, (c, wd)), jnp.broadcast_to(t, (c, wd))


def kernel(feats, shared_w, shared_bn_g, shared_bn_b,
           reg_w1, reg_bn_g, reg_bn_b, reg_w2, reg_b2,
           height_w1, height_bn_g, height_bn_b, height_w2, height_b2,
           dim_w1, dim_bn_g, dim_bn_b, dim_w2, dim_b2,
           rot_w1, rot_bn_g, rot_bn_b, rot_w2, rot_b2,
           vel_w1, vel_bn_g, vel_bn_b, vel_w2, vel_b2,
           heatmap_w1, heatmap_bn_g, heatmap_bn_b, heatmap_w2, heatmap_b2):
    x = jnp.transpose(feats[0], (0, 2, 1, 3)).astype(jnp.bfloat16)
    B, nh, _, wd = x.shape
    n = B * nh * wd

    y, s1, q1 = _conv(x, _prep_w(shared_w), with_stats=True,
                      out_dtype=jnp.bfloat16)
    sc1, tc1 = _bn_fold(s1, q1, shared_bn_g, shared_bn_b, n, wd)

    w1 = jnp.concatenate(
        [reg_w1, height_w1, dim_w1, rot_w1, vel_w1, heatmap_w1], axis=0)
    hh, s2, q2 = _conv(y, _prep_w(w1), s=sc1, t=tc1, with_stats=True,
                       out_dtype=jnp.bfloat16)
    g2 = jnp.concatenate(
        [reg_bn_g, height_bn_g, dim_bn_g, rot_bn_g, vel_bn_g, heatmap_bn_g])
    bb2 = jnp.concatenate(
        [reg_bn_b, height_bn_b, dim_bn_b, rot_bn_b, vel_bn_b, heatmap_bn_b])
    sc2, tc2 = _bn_fold(s2, q2, g2, bb2, n, wd)

    ocs = (2, 1, 3, 2, 2, 1)
    w2s = (reg_w2, height_w2, dim_w2, rot_w2, vel_w2, heatmap_w2)
    b2s = (reg_b2, height_b2, dim_b2, rot_b2, vel_b2, heatmap_b2)
    c1 = w2s[0].shape[1]
    wbd = jnp.zeros((16, 6 * c1, 3, 3), jnp.float32)
    r = 0
    for k, (oc, w2) in enumerate(zip(ocs, w2s)):
        wbd = wbd.at[r:r + oc, c1 * k:c1 * (k + 1)].set(w2)
        r += oc
    bias = jnp.pad(jnp.concatenate(b2s).reshape(-1, 1), ((0, 16 - r), (0, 0)))
    bias = jnp.broadcast_to(bias, (16, wd))

    out3 = _conv(hh, _prep_w(wbd), s=sc2, t=tc2, bias=bias)[0]
    out3 = jnp.transpose(out3, (0, 2, 1, 3))

    outs = []
    r = 0
    for oc in ocs:
        outs.append(out3[:, r:r + oc])
        r += oc
    return tuple(outs)


# speedup vs baseline: 4.3709x; 1.1148x over previous
import functools

import jax
import jax.numpy as jnp
from jax.experimental import pallas as pl
from jax.experimental.pallas import tpu as pltpu

_EPS = 1e-5
_ROWS = 32


def _shift(v, s, blk):
    if s == 0:
        return v
    rolled = jnp.roll(v, s, axis=1)
    lane = jnp.bitwise_and(
        jax.lax.broadcasted_iota(jnp.int32, v.shape, 1), blk - 1)
    if s > 0:
        return jnp.where(lane < s, 0.0, rolled)
    return jnp.where(lane >= blk + s, 0.0, rolled)


def _conv_body(*refs, fuse_bn, with_bias, with_stats, nh, rows):
    x_refs = refs[0:3]
    w_ref = refs[3]
    i = 4
    if fuse_bn:
        s_ref, t_ref = refs[i], refs[i + 1]
        i += 2
    if with_bias:
        b_ref = refs[i]
        i += 1
    o_ref = refs[i]
    i += 1
    if with_stats:
        sum_ref, sq_ref = refs[i], refs[i + 1]

    b = pl.program_id(0)
    h = pl.program_id(1)

    if with_stats:
        @pl.when(jnp.logical_and(b == 0, h == 0))
        def _init():
            sum_ref[...] = jnp.zeros_like(sum_ref)
            sq_ref[...] = jnp.zeros_like(sq_ref)

    wd = o_ref.shape[3]
    slabs = []
    for L in range(rows - 1, 2 * rows + 1):
        v = x_refs[L // rows][0, L % rows, :, :]
        if fuse_bn:
            v = jnp.maximum(v.astype(jnp.float32) * s_ref[...] + t_ref[...],
                            0.0)
        gi = h * rows + (L - rows)
        valid = jnp.logical_and(gi >= 0, gi < nh)
        v = v * jnp.where(valid, 1.0, 0.0).astype(v.dtype)
        slabs.append(v.astype(jnp.bfloat16))
    slab = jnp.concatenate(slabs, axis=1)

    xk = jnp.concatenate(
        [slab[:, d * wd:(d + rows) * wd] for d in range(3)], axis=0)
    p = jax.lax.dot(w_ref[...], xk, preferred_element_type=jnp.float32)
    cout = o_ref.shape[2]
    acc = (_shift(p[0:cout], 1, wd) + p[cout:2 * cout]
           + _shift(p[2 * cout:3 * cout], -1, wd))
    if with_bias:
        acc = acc + jnp.tile(b_ref[...], (1, rows))
    for r in range(rows):
        o_ref[0, r, :, :] = acc[:, r * wd:(r + 1) * wd].astype(o_ref.dtype)
    if with_stats:
        sum_ref[...] += acc
        sq_ref[...] += acc * acc


def _xim(b, h, *, j, nt):
    return (b, jnp.clip(h + j - 1, 0, nt - 1), 0, 0)


def _conv(x, wcat, s=None, t=None, bias=None, with_stats=False,
          out_dtype=jnp.float32):
    B, nh, cin, wd = x.shape
    cout = wcat.shape[0] // 3
    fuse_bn = s is not None
    with_bias = bias is not None
    rows = _ROWS
    nt = nh // rows

    in_specs = [
        pl.BlockSpec((1, rows, cin, wd), functools.partial(_xim, j=j, nt=nt))
        for j in range(3)
    ]
    in_specs.append(pl.BlockSpec(wcat.shape, lambda b, h: (0, 0)))
    operands = [x, x, x, wcat]
    if fuse_bn:
        in_specs += [pl.BlockSpec(s.shape, lambda b, h: (0, 0)),
                     pl.BlockSpec(t.shape, lambda b, h: (0, 0))]
        operands += [s, t]
    if with_bias:
        in_specs.append(pl.BlockSpec(bias.shape, lambda b, h: (0, 0)))
        operands.append(bias)

    out_shape = [jax.ShapeDtypeStruct((B, nh, cout, wd), out_dtype)]
    out_specs = [pl.BlockSpec((1, rows, cout, wd), lambda b, h: (b, h, 0, 0))]
    if with_stats:
        out_shape += [jax.ShapeDtypeStruct((cout, rows * wd), jnp.float32)] * 2
        out_specs += [pl.BlockSpec((cout, rows * wd), lambda b, h: (0, 0))] * 2

    body = functools.partial(_conv_body, fuse_bn=fuse_bn, with_bias=with_bias,
                             with_stats=with_stats, nh=nh, rows=rows)
    res = pl.pallas_call(
        body,
        grid=(B, nt),
        in_specs=in_specs,
        out_specs=out_specs,
        out_shape=out_shape,
        compiler_params=pltpu.CompilerParams(
            dimension_semantics=("arbitrary", "arbitrary")),
    )(*operands)
    return res


def _prep_w(w):
    wt = jnp.transpose(w, (3, 0, 2, 1)).reshape(3 * w.shape[0],
                                                3 * w.shape[1])
    return wt.astype(jnp.bfloat16)


def _bn_fold(sum_o, sq_o, g, b, n, wd):
    m = jnp.sum(sum_o, axis=1, keepdims=True) / n
    v = jnp.sum(sq_o, axis=1, keepdims=True) / n - m * m
    s = g.reshape(-1, 1) * jax.lax.rsqrt(v + _EPS)
    t = b.reshape(-1, 1) - m * s
    c = s.shape[0]
    return jnp.broadcast_to(s, (c, wd)), jnp.broadcast_to(t, (c, wd))


def kernel(feats, shared_w, shared_bn_g, shared_bn_b,
           reg_w1, reg_bn_g, reg_bn_b, reg_w2, reg_b2,
           height_w1, height_bn_g, height_bn_b, height_w2, height_b2,
           dim_w1, dim_bn_g, dim_bn_b, dim_w2, dim_b2,
           rot_w1, rot_bn_g, rot_bn_b, rot_w2, rot_b2,
           vel_w1, vel_bn_g, vel_bn_b, vel_w2, vel_b2,
           heatmap_w1, heatmap_bn_g, heatmap_bn_b, heatmap_w2, heatmap_b2):
    x = jnp.transpose(feats[0], (0, 2, 1, 3)).astype(jnp.bfloat16)
    B, nh, _, wd = x.shape
    n = B * nh * wd

    y, s1, q1 = _conv(x, _prep_w(shared_w), with_stats=True,
                      out_dtype=jnp.bfloat16)
    sc1, tc1 = _bn_fold(s1, q1, shared_bn_g, shared_bn_b, n, wd)

    w1 = jnp.concatenate(
        [reg_w1, height_w1, dim_w1, rot_w1, vel_w1, heatmap_w1], axis=0)
    hh, s2, q2 = _conv(y, _prep_w(w1), s=sc1, t=tc1, with_stats=True,
                       out_dtype=jnp.bfloat16)
    g2 = jnp.concatenate(
        [reg_bn_g, height_bn_g, dim_bn_g, rot_bn_g, vel_bn_g, heatmap_bn_g])
    bb2 = jnp.concatenate(
        [reg_bn_b, height_bn_b, dim_bn_b, rot_bn_b, vel_bn_b, heatmap_bn_b])
    sc2, tc2 = _bn_fold(s2, q2, g2, bb2, n, wd)

    ocs = (2, 1, 3, 2, 2, 1)
    w2s = (reg_w2, height_w2, dim_w2, rot_w2, vel_w2, heatmap_w2)
    b2s = (reg_b2, height_b2, dim_b2, rot_b2, vel_b2, heatmap_b2)
    c1 = w2s[0].shape[1]
    wbd = jnp.zeros((16, 6 * c1, 3, 3), jnp.float32)
    r = 0
    for k, (oc, w2) in enumerate(zip(ocs, w2s)):
        wbd = wbd.at[r:r + oc, c1 * k:c1 * (k + 1)].set(w2)
        r += oc
    bias = jnp.pad(jnp.concatenate(b2s).reshape(-1, 1), ((0, 16 - r), (0, 0)))
    bias = jnp.broadcast_to(bias, (16, wd))

    out3 = _conv(hh, _prep_w(wbd), s=sc2, t=tc2, bias=bias)[0]
    out3 = jnp.transpose(out3, (0, 2, 1, 3))

    outs = []
    r = 0
    for oc in ocs:
        outs.append(out3[:, r:r + oc])
        r += oc
    return tuple(outs)
